# baseline (device time: 74014 ns/iter reference)
import jax
import jax.numpy as jnp
from jax import lax
from jax.experimental import pallas as pl
from jax.experimental.pallas import tpu as pltpu

N_DEV = 4
B, SQ, SKV, DH = 2, 512, 512, 64
H_LOC = 8
D_LOC = H_LOC * DH
D_MODEL = 768
BLK = 64


def kernel(x, Wq, K_ext, V_ext, Wo):
    def body(x_ref, wq_ref, k_ref, v_ref, wo_ref, out_ref,
             comm_ref, send_sems, recv_sems):
        my = lax.axis_index("i")
        left = lax.rem(my + N_DEV - 1, N_DEV)
        right = lax.rem(my + 1, N_DEV)

        barrier = pltpu.get_barrier_semaphore()
        for nbr in (left, right):
            pl.semaphore_signal(barrier, inc=1, device_id=(nbr,),
                                device_id_type=pl.DeviceIdType.MESH)
        pl.semaphore_wait(barrier, 2)

        xf = x_ref[...].reshape(B * SQ, D_MODEL).astype(jnp.bfloat16)
        wq = wq_ref[:, pl.ds(my * D_LOC, D_LOC)].astype(jnp.bfloat16)
        q = lax.dot_general(xf, wq, (((1,), (0,)), ((), ())),
                            preferred_element_type=jnp.float32)
        q = (q * 0.125).astype(jnp.bfloat16)

        qb = lax.broadcasted_iota(jnp.int32, (SQ, SKV), 0) // BLK
        kb = lax.broadcasted_iota(jnp.int32, (SQ, SKV), 1) // BLK
        mask = (qb == kb) | (kb == 0) | (lax.rem(qb + kb, 3) == 0)
        bias = jnp.where(mask, 0.0, -1e9)

        for b in range(B):
            for h in range(H_LOC):
                q_bh = q[b * SQ:(b + 1) * SQ, h * DH:(h + 1) * DH]
                k_bh = k_ref[b, :, h, :].astype(jnp.bfloat16)
                v_bh = v_ref[b, :, h, :].astype(jnp.bfloat16)
                s = lax.dot_general(q_bh, k_bh, (((1,), (1,)), ((), ())),
                                    preferred_element_type=jnp.float32)
                s = s + bias
                m = jnp.max(s, axis=1, keepdims=True)
                w = jnp.exp(s - m)
                w = (w / jnp.sum(w, axis=1, keepdims=True)).astype(jnp.bfloat16)
                ctx = lax.dot_general(w, v_bh, (((1,), (0,)), ((), ())),
                                      preferred_element_type=jnp.float32)
                comm_ref[0, b * SQ:(b + 1) * SQ, h * DH:(h + 1) * DH] = (
                    ctx.astype(jnp.bfloat16))

        wo_my = wo_ref[pl.ds(my * D_LOC, D_LOC), :].astype(jnp.bfloat16)
        acc = lax.dot_general(comm_ref[0], wo_my, (((1,), (0,)), ((), ())),
                              preferred_element_type=jnp.float32)

        for hop in range(N_DEV - 1):
            rdma = pltpu.make_async_remote_copy(
                src_ref=comm_ref.at[hop],
                dst_ref=comm_ref.at[hop + 1],
                send_sem=send_sems.at[hop],
                recv_sem=recv_sems.at[hop],
                device_id=(right,),
                device_id_type=pl.DeviceIdType.MESH,
            )
            rdma.start()
            rdma.wait()
            origin = lax.rem(my + N_DEV - 1 - hop, N_DEV)
            wo_o = wo_ref[pl.ds(origin * D_LOC, D_LOC), :].astype(jnp.bfloat16)
            acc = acc + lax.dot_general(
                comm_ref[hop + 1], wo_o, (((1,), (0,)), ((), ())),
                preferred_element_type=jnp.float32)

        out_ref[...] = acc.reshape(B, SQ, D_MODEL)

    return pl.pallas_call(
        body,
        out_shape=jax.ShapeDtypeStruct((B, SQ, D_MODEL), jnp.float32),
        in_specs=[pl.BlockSpec(memory_space=pltpu.VMEM)] * 5,
        out_specs=pl.BlockSpec(memory_space=pltpu.VMEM),
        scratch_shapes=[
            pltpu.VMEM((N_DEV, B * SQ, D_LOC), jnp.bfloat16),
            pltpu.SemaphoreType.DMA((N_DEV - 1,)),
            pltpu.SemaphoreType.DMA((N_DEV - 1,)),
        ],
        compiler_params=pltpu.CompilerParams(collective_id=0),
    )(x, Wq, K_ext, V_ext, Wo)


# device time: 32162 ns/iter; 2.3013x vs baseline; 2.3013x over previous
import jax
import jax.numpy as jnp
from jax import lax
from jax.experimental import pallas as pl
from jax.experimental.pallas import tpu as pltpu

N_DEV = 4
B, SQ, SKV, DH = 2, 512, 512, 64
H_LOC = 8
D_LOC = H_LOC * DH
HALF = D_LOC // 2
D_MODEL = 768
BLK = 64


def kernel(x, Wq, K_ext, V_ext, Wo):
    def body(x_ref, wq_ref, k_ref, v_ref, wo_ref, out_ref,
             comm0, comm1, s0, r0, s1, r1):
        my = lax.axis_index("i")
        left = lax.rem(my + N_DEV - 1, N_DEV)
        right = lax.rem(my + 1, N_DEV)

        barrier = pltpu.get_barrier_semaphore()
        for nbr in (left, right):
            pl.semaphore_signal(barrier, inc=1, device_id=(nbr,),
                                device_id_type=pl.DeviceIdType.MESH)
        pl.semaphore_wait(barrier, 2)

        xf = x_ref[...].reshape(B * SQ, D_MODEL).astype(jnp.bfloat16)
        wq = wq_ref[:, pl.ds(my * D_LOC, D_LOC)].astype(jnp.bfloat16)
        q = lax.dot_general(xf, wq, (((1,), (0,)), ((), ())),
                            preferred_element_type=jnp.float32)
        q = (q * 0.125).astype(jnp.bfloat16)

        qb = lax.broadcasted_iota(jnp.int32, (SQ, SKV), 0) // BLK
        kb = lax.broadcasted_iota(jnp.int32, (SQ, SKV), 1) // BLK
        mask = (qb == kb) | (kb == 0) | (lax.rem(qb + kb, 3) == 0)
        bias = jnp.where(mask, 0.0, -1e9)

        for b in range(B):
            for h in range(H_LOC):
                q_bh = q[b * SQ:(b + 1) * SQ, h * DH:(h + 1) * DH]
                k_bh = k_ref[b, :, h, :].astype(jnp.bfloat16)
                v_bh = v_ref[b, :, h, :].astype(jnp.bfloat16)
                s = lax.dot_general(q_bh, k_bh, (((1,), (1,)), ((), ())),
                                    preferred_element_type=jnp.float32)
                w = jnp.exp(s + bias)
                denom = jnp.sum(w, axis=1, keepdims=True)
                ctx = lax.dot_general(w.astype(jnp.bfloat16), v_bh,
                                      (((1,), (0,)), ((), ())),
                                      preferred_element_type=jnp.float32)
                ctx = (ctx * (1.0 / denom)).astype(jnp.bfloat16)
                dst = comm0 if h < 4 else comm1
                c = (h % 4) * DH
                dst[0, b * SQ:(b + 1) * SQ, c:c + DH] = ctx

        def hop_rdma(hop):
            f = pltpu.make_async_remote_copy(
                src_ref=comm0.at[hop], dst_ref=comm0.at[hop + 1],
                send_sem=s0.at[hop], recv_sem=r0.at[hop],
                device_id=(right,), device_id_type=pl.DeviceIdType.MESH)
            g = pltpu.make_async_remote_copy(
                src_ref=comm1.at[hop], dst_ref=comm1.at[hop + 1],
                send_sem=s1.at[hop], recv_sem=r1.at[hop],
                device_id=(left,), device_id_type=pl.DeviceIdType.MESH)
            return f, g

        def wo_half(col_start):
            return wo_ref[pl.ds(col_start, HALF), :].astype(jnp.bfloat16)

        def half_dot(chunk, wo_slice):
            return lax.dot_general(chunk, wo_slice, (((1,), (0,)), ((), ())),
                                   preferred_element_type=jnp.float32)

        rdmas = [hop_rdma(0)]
        rdmas[0][0].start()
        rdmas[0][1].start()
        acc = half_dot(comm0[0], wo_half(my * D_LOC))
        acc = acc + half_dot(comm1[0], wo_half(my * D_LOC + HALF))

        for hop in range(N_DEV - 1):
            f, g = rdmas[hop]
            f.wait_recv()
            g.wait_recv()
            if hop < N_DEV - 2:
                nf, ng = hop_rdma(hop + 1)
                nf.start()
                ng.start()
                rdmas.append((nf, ng))
            o0 = lax.rem(my + N_DEV - 1 - hop, N_DEV)
            o1 = lax.rem(my + 1 + hop, N_DEV)
            acc = acc + half_dot(comm0[hop + 1], wo_half(o0 * D_LOC))
            acc = acc + half_dot(comm1[hop + 1], wo_half(o1 * D_LOC + HALF))
            f.wait_send()
            g.wait_send()

        out_ref[...] = acc.reshape(B, SQ, D_MODEL)

    return pl.pallas_call(
        body,
        out_shape=jax.ShapeDtypeStruct((B, SQ, D_MODEL), jnp.float32),
        in_specs=[pl.BlockSpec(memory_space=pltpu.VMEM)] * 5,
        out_specs=pl.BlockSpec(memory_space=pltpu.VMEM),
        scratch_shapes=[
            pltpu.VMEM((N_DEV, B * SQ, HALF), jnp.bfloat16),
            pltpu.VMEM((N_DEV, B * SQ, HALF), jnp.bfloat16),
            pltpu.SemaphoreType.DMA((N_DEV - 1,)),
            pltpu.SemaphoreType.DMA((N_DEV - 1,)),
            pltpu.SemaphoreType.DMA((N_DEV - 1,)),
            pltpu.SemaphoreType.DMA((N_DEV - 1,)),
        ],
        compiler_params=pltpu.CompilerParams(collective_id=0),
    )(x, Wq, K_ext, V_ext, Wo)
